# baseline (speedup 1.0000x reference)
import functools

import jax
import jax.numpy as jnp
from jax.experimental import pallas as pl
from jax.experimental.pallas import tpu as pltpu
from jax.experimental.pallas import tpu_sc as plsc

B, S, H = 8, 32, 1024
NH, NKV, HD = 16, 8, 64
G = NH // NKV
E, TOPK, CAP = 64, 8, 64
FF = 6144
FH = FF // 2
T = B * S
NSLOT = E * CAP


def _rms(x, w, eps=1e-6):
    v = jnp.mean(jnp.square(x), axis=-1, keepdims=True)
    return w * (x * jax.lax.rsqrt(v + eps))


def _rot_half(x):
    h = x.shape[-1] // 2
    return jnp.concatenate([-x[:, h:], x[:, :h]], axis=-1)


HI = jax.lax.Precision.HIGHEST
HP = jax.lax.Precision.HIGHEST


def _attn_kernel(hs_ref, cos_ref, sin_ref, mask_ref, win_ref, wpost_ref,
                 qln_ref, kln_ref, qkv_w_ref, o_w_ref, res2_ref, flat_ref):
    x = hs_ref[...]
    xn = _rms(x, win_ref[...])
    qkv = jnp.dot(xn, qkv_w_ref[...], preferred_element_type=jnp.float32,
                  precision=HP)
    cos = cos_ref[...]
    sin = sin_ref[...]
    mask = mask_ref[...]
    qln = qln_ref[...]
    kln = kln_ref[...]

    ks = []
    vs = []
    for kv in range(NKV):
        base = kv * (G + 2) * HD
        k = qkv[:, base + G * HD: base + (G + 1) * HD]
        v = qkv[:, base + (G + 1) * HD: base + (G + 2) * HD]
        k = k * cos + _rot_half(k) * sin
        k = _rms(k, kln)
        ks.append(k)
        vs.append(v)

    mask2 = jnp.concatenate([mask, mask], axis=0)
    outs = []
    for kv in range(NKV):
        base = kv * (G + 2) * HD
        qs = []
        for g in range(G):
            q = qkv[:, base + g * HD: base + (g + 1) * HD]
            q = q * cos + _rot_half(q) * sin
            qs.append(_rms(q, qln))
        qpair = jnp.concatenate(qs, axis=0)
        s = jax.lax.dot_general(qpair, ks[kv], (((1,), (1,)), ((), ())),
                                preferred_element_type=jnp.float32,
                                precision=HP)
        s = s * (1.0 / 8.0) + mask2
        m = jnp.max(s, axis=-1, keepdims=True)
        e = jnp.exp(s - m)
        a = e / jnp.sum(e, axis=-1, keepdims=True)
        av = jnp.dot(a, vs[kv], preferred_element_type=jnp.float32,
                     precision=HP)
        for g in range(G):
            outs.append(av[g * T:(g + 1) * T, :])

    cat = jnp.concatenate(outs, axis=-1)
    o = jnp.dot(cat, o_w_ref[...], preferred_element_type=jnp.float32,
                precision=HP)
    res2 = x + o
    res2_ref[...] = res2
    flat_ref[...] = _rms(res2, wpost_ref[...])


def _attention(hidden_states, cos, sin, mask, win, wpost, qln, kln, qkv_w, o_w):
    out_shape = [jax.ShapeDtypeStruct((T, H), jnp.float32),
                 jax.ShapeDtypeStruct((T, H), jnp.float32)]
    full = lambda shape: pl.BlockSpec(shape, lambda: tuple(0 for _ in shape))
    return pl.pallas_call(
        _attn_kernel,
        in_specs=[
            full((T, H)), full((T, HD)), full((T, HD)), full((T, T)),
            full((1, H)), full((1, H)), full((1, HD)), full((1, HD)),
            full((H, HD * (NH + 2 * NKV))), full((NH * HD, H)),
        ],
        out_specs=[full((T, H)), full((T, H))],
        out_shape=out_shape,
    )(hidden_states, cos, sin, mask, win, wpost, qln, kln, qkv_w, o_w)


def _routing_kernel(flat_ref, wg_ref, d_ref, c_ref):
    flat = flat_ref[...]
    logits = jnp.dot(flat, wg_ref[...], preferred_element_type=jnp.float32,
                     precision=jax.lax.Precision.HIGHEST)
    m = jnp.max(logits, axis=-1, keepdims=True)
    ex = jnp.exp(logits - m)
    gates = ex / jnp.sum(ex, axis=-1, keepdims=True)

    lane_e = jax.lax.broadcasted_iota(jnp.int32, (T, E), 1)
    work = gates
    idx_cols = []
    w_cols = []
    for _ in range(TOPK):
        mx = jnp.max(work, axis=-1, keepdims=True)
        eq = work == mx
        idx = jnp.min(jnp.where(eq, lane_e, E + 1), axis=-1, keepdims=True)
        idx_cols.append(idx)
        w_cols.append(mx)
        work = jnp.where(lane_e == idx, -1e30, work)

    wsum = w_cols[0]
    for j in range(1, TOPK):
        wsum = wsum + w_cols[j]
    denom = jnp.maximum(wsum, 1.1920929e-07)
    w_cols = [w / denom for w in w_cols]

    cnt = jnp.zeros((T, E), jnp.float32)
    ohs = []
    for j in range(TOPK):
        oh = (lane_e == idx_cols[j]).astype(jnp.float32)
        ohs.append(oh)
        cnt = cnt + oh
    r = jax.lax.broadcasted_iota(jnp.int32, (T, T), 0)
    c = jax.lax.broadcasted_iota(jnp.int32, (T, T), 1)
    lstrict = (r > c).astype(jnp.float32)
    base_excl = jnp.dot(lstrict, cnt, preferred_element_type=jnp.float32)

    slot_cols = []
    for j in range(TOPK):
        intra = jnp.zeros((T, 1), jnp.float32)
        for j2 in range(j):
            intra = intra + (idx_cols[j2] == idx_cols[j]).astype(jnp.float32)
        pos = jnp.sum(base_excl * ohs[j], axis=-1, keepdims=True) + intra
        slot_cols.append(jnp.where(
            pos < CAP, idx_cols[j].astype(jnp.float32) * CAP + pos,
            float(NSLOT)))

    slot = jnp.concatenate(slot_cols, axis=-1)
    wk = jnp.concatenate(w_cols, axis=-1)

    i8r = jax.lax.broadcasted_iota(jnp.int32, (TOPK, TOPK), 0)
    i8c = jax.lax.broadcasted_iota(jnp.int32, (TOPK, TOPK), 1)
    eye8 = (i8r == i8c).astype(jnp.float32)
    tr = lambda a: jax.lax.dot_general(eye8, a, (((1,), (1,)), ((), ())),
                                       preferred_element_type=jnp.float32,
                                       precision=jax.lax.Precision.HIGHEST)
    d_ref[...] = tr(slot)
    c_ref[...] = tr(wk)


def _routing(flat, wg):
    out_shape = [jax.ShapeDtypeStruct((TOPK, T), jnp.float32),
                 jax.ShapeDtypeStruct((TOPK, T), jnp.float32)]
    return pl.pallas_call(
        _routing_kernel,
        in_specs=[pl.BlockSpec((T, H), lambda: (0, 0)),
                  pl.BlockSpec((H, E), lambda: (0, 0))],
        out_specs=[pl.BlockSpec((TOPK, T), lambda: (0, 0)),
                   pl.BlockSpec((TOPK, T), lambda: (0, 0))],
        out_shape=out_shape,
    )(flat, wg)


NSF = 4
SFB = FH // NSF


def _shared_kernel(flat_ref, res2_ref, gu1_ref, gu2_ref, dw_ref, base_ref):
    fs = pl.program_id(0)
    flat = flat_ref[...]
    g1 = jnp.dot(flat, gu1_ref[...], preferred_element_type=jnp.float32)
    g2 = jnp.dot(flat, gu2_ref[...], preferred_element_type=jnp.float32)
    act = g1 * (g2 * jax.nn.sigmoid(g2))
    y = jnp.dot(act, dw_ref[...], preferred_element_type=jnp.float32)

    @pl.when(fs == 0)
    def _():
        base_ref[...] = res2_ref[...] + y

    @pl.when(fs != 0)
    def _():
        base_ref[...] = base_ref[...] + y


def _shared(flat, res2, shared_gu, shared_dw):
    return pl.pallas_call(
        _shared_kernel,
        grid=(NSF,),
        in_specs=[
            pl.BlockSpec((T, H), lambda fs: (0, 0)),
            pl.BlockSpec((T, H), lambda fs: (0, 0)),
            pl.BlockSpec((H, SFB), lambda fs: (0, fs)),
            pl.BlockSpec((H, SFB), lambda fs: (0, fs + NSF)),
            pl.BlockSpec((SFB, H), lambda fs: (fs, 0)),
        ],
        out_specs=pl.BlockSpec((T, H), lambda fs: (0, 0)),
        out_shape=jax.ShapeDtypeStruct((T, H), jnp.float32),
    )(flat, res2, shared_gu, shared_gu, shared_dw)


NSRC = NSLOT + 128
W = 128

_SCALAR_MESH = plsc.ScalarSubcoreMesh(axis_name="core", num_cores=2)
_VECTOR_MESH = plsc.VectorSubcoreMesh(core_axis_name="core",
                                      subcore_axis_name="subcore")


def _sc_src(slot_i, tok):
    @pl.kernel(out_type=jax.ShapeDtypeStruct((1, NSRC), jnp.int32),
               mesh=_SCALAR_MESH,
               scratch_types=[pltpu.SMEM((NSRC,), jnp.int32),
                              pltpu.SMEM((T * TOPK,), jnp.int32),
                              pltpu.SMEM((T * TOPK,), jnp.int32),
                              pltpu.SemaphoreType.DMA])
    def ka(slot_hbm, tok_hbm, src_hbm, src_s, slot_s, tok_s, sem):
        idx = jax.lax.axis_index("core")

        @pl.when(idx == 0)
        def _():
            pltpu.async_copy(slot_hbm.at[0], slot_s, sem).wait()
            pltpu.async_copy(tok_hbm.at[0], tok_s, sem).wait()

            @pl.loop(0, NSRC)
            def _(i):
                src_s[i] = T

            @pl.loop(0, T * TOPK)
            def _(i):
                src_s[slot_s[i]] = tok_s[i]

            pltpu.async_copy(src_s, src_hbm.at[0], sem).wait()

    return ka(slot_i, tok)


NQ = 4
HQ = H // NQ


def _sc_gather(flat_qs, src):
    out_types = [jax.ShapeDtypeStruct((NSRC, HQ), jnp.float32)
                 for _ in range(NQ)]

    @pl.kernel(out_type=out_types, mesh=_VECTOR_MESH)
    def kb(f0, f1, f2, f3, src_hbm, b0, b1, b2, b3):
        for fq, bq in ((f0, b0), (f1, b1), (f2, b2), (f3, b3)):
            def body(i_vmem, o_vmem, fq=fq):
                pltpu.sync_copy(fq.at[i_vmem.at[0]], o_vmem)

            pltpu.emit_pipeline(
                body,
                grid=(NSRC // W,),
                in_specs=[pl.BlockSpec((1, W), lambda i: (0, i))],
                out_specs=[pl.BlockSpec((W, HQ), lambda i: (i, 0))],
                core_axis_name=("core", "subcore"),
                dimension_semantics=(pltpu.PARALLEL,),
            )(src_hbm, bq)

    return kb(*flat_qs, src)


NF = 2
FB = FH // NF


def _moe_kernel(slot_ref, wk_ref, b0_ref, b1_ref, b2_ref, b3_ref, base_ref,
                gu1_ref, gu2_ref, dw_ref, out_ref, c_ref):
    e = pl.program_id(0)
    f = pl.program_id(1)

    @pl.when(f == 0)
    def _():
        st = slot_ref[...].astype(jnp.int32) - e * CAP
        wk = wk_ref[...]
        ci = jax.lax.broadcasted_iota(jnp.int32, (CAP, T), 0)
        cm = jnp.zeros((CAP, T), jnp.float32)
        for j in range(TOPK):
            eqf = (ci == st[j:j + 1, :]).astype(jnp.float32)
            cm = cm + eqf * wk[j:j + 1, :]
        c_ref[...] = cm

    x = jnp.concatenate([b0_ref[...], b1_ref[...], b2_ref[...], b3_ref[...]],
                        axis=1)
    g1 = jnp.dot(x, gu1_ref[0], preferred_element_type=jnp.float32)
    g2 = jnp.dot(x, gu2_ref[0], preferred_element_type=jnp.float32)
    act = g1 * (g2 * jax.nn.sigmoid(g2))
    y = jnp.dot(act, dw_ref[0], preferred_element_type=jnp.float32)
    contrib = jax.lax.dot_general(c_ref[...], y, (((0,), (0,)), ((), ())),
                                  preferred_element_type=jnp.float32)

    @pl.when((e == 0) & (f == 0))
    def _():
        out_ref[...] = base_ref[...] + contrib

    @pl.when((e != 0) | (f != 0))
    def _():
        out_ref[...] = out_ref[...] + contrib


def _moe(slot_t, wk_t, bufs, base, expert_gu, expert_dw):
    return pl.pallas_call(
        _moe_kernel,
        grid=(E, NF),
        in_specs=[
            pl.BlockSpec((TOPK, T), lambda e, f: (0, 0)),
            pl.BlockSpec((TOPK, T), lambda e, f: (0, 0)),
            pl.BlockSpec((CAP, HQ), lambda e, f: (e, 0)),
            pl.BlockSpec((CAP, HQ), lambda e, f: (e, 0)),
            pl.BlockSpec((CAP, HQ), lambda e, f: (e, 0)),
            pl.BlockSpec((CAP, HQ), lambda e, f: (e, 0)),
            pl.BlockSpec((T, H), lambda e, f: (0, 0)),
            pl.BlockSpec((1, H, FB), lambda e, f: (e, 0, f)),
            pl.BlockSpec((1, H, FB), lambda e, f: (e, 0, f + NF)),
            pl.BlockSpec((1, FB, H), lambda e, f: (e, f, 0)),
        ],
        out_specs=pl.BlockSpec((T, H), lambda e, f: (0, 0)),
        out_shape=jax.ShapeDtypeStruct((T, H), jnp.float32),
        scratch_shapes=[pltpu.VMEM((CAP, T), jnp.float32)],
    )(slot_t, wk_t, *bufs, base, expert_gu, expert_gu, expert_dw)


@jax.jit
def kernel(hidden_states, cos, sin, attention_mask, input_ln_w, post_ln_w,
           q_ln_w, k_ln_w, qkv_w, o_w, wg, shared_gu, shared_dw,
           expert_gu, expert_dw):
    cos_t = jnp.tile(cos[0], (B, 1))
    sin_t = jnp.tile(sin[0], (B, 1))
    ri = jnp.arange(T)
    same_batch = (ri[:, None] // S) == (ri[None, :] // S)
    mask_full = jnp.where(same_batch, jnp.tile(attention_mask[0, 0], (B, B)),
                          -1e9).astype(jnp.float32)
    res2, flat = _attention(
        hidden_states.reshape(T, H), cos_t, sin_t, mask_full,
        input_ln_w.reshape(1, H), post_ln_w.reshape(1, H),
        q_ln_w.reshape(1, HD), k_ln_w.reshape(1, HD), qkv_w, o_w)
    slot_t, wk_t = _routing(flat, wg)
    slot_i = slot_t.astype(jnp.int32).reshape(1, T * TOPK)
    tok = jnp.broadcast_to(jnp.arange(T, dtype=jnp.int32)[None, :],
                           (TOPK, T)).reshape(1, T * TOPK)
    src = _sc_src(slot_i, tok)
    flat_ext = jnp.concatenate([flat, jnp.zeros((8, H), jnp.float32)], axis=0)
    flat_qs = [flat_ext[:, q * HQ:(q + 1) * HQ] for q in range(NQ)]
    bufs = _sc_gather(flat_qs, src)
    base = _shared(flat, res2, shared_gu, shared_dw)
    out = _moe(slot_t, wk_t, bufs, base, expert_gu, expert_dw)
    return out.reshape(B, S, H)


# speedup vs baseline: 1.3971x; 1.3971x over previous
import functools

import jax
import jax.numpy as jnp
from jax.experimental import pallas as pl
from jax.experimental.pallas import tpu as pltpu

B, S, H = 8, 32, 1024
NH, NKV, HD = 16, 8, 64
G = NH // NKV
E, TOPK, CAP = 64, 8, 64
FF = 6144
FH = FF // 2
T = B * S
NSLOT = E * CAP


def _rms(x, w, eps=1e-6):
    v = jnp.mean(jnp.square(x), axis=-1, keepdims=True)
    return w * (x * jax.lax.rsqrt(v + eps))


def _rot_half(x):
    h = x.shape[-1] // 2
    return jnp.concatenate([-x[:, h:], x[:, :h]], axis=-1)


HI = jax.lax.Precision.HIGHEST
HP = jax.lax.Precision.DEFAULT


def _attn_kernel(hs_ref, cos_ref, sin_ref, mask_ref, win_ref, wpost_ref,
                 qln_ref, kln_ref, qkv_w_ref, o_w_ref, res2_ref, flat_ref):
    x = hs_ref[...]
    xn = _rms(x, win_ref[...])
    qkv = jnp.dot(xn, qkv_w_ref[...], preferred_element_type=jnp.float32,
                  precision=HP)
    cos = cos_ref[...]
    sin = sin_ref[...]
    mask = mask_ref[...]
    qln = qln_ref[...]
    kln = kln_ref[...]

    ks = []
    vs = []
    for kv in range(NKV):
        base = kv * (G + 2) * HD
        k = qkv[:, base + G * HD: base + (G + 1) * HD]
        v = qkv[:, base + (G + 1) * HD: base + (G + 2) * HD]
        k = k * cos + _rot_half(k) * sin
        k = _rms(k, kln)
        ks.append(k)
        vs.append(v)

    mask2 = jnp.concatenate([mask, mask], axis=0)
    outs = []
    for kv in range(NKV):
        base = kv * (G + 2) * HD
        qs = []
        for g in range(G):
            q = qkv[:, base + g * HD: base + (g + 1) * HD]
            q = q * cos + _rot_half(q) * sin
            qs.append(_rms(q, qln))
        qpair = jnp.concatenate(qs, axis=0)
        s = jax.lax.dot_general(qpair, ks[kv], (((1,), (1,)), ((), ())),
                                preferred_element_type=jnp.float32,
                                precision=HP)
        s = s * (1.0 / 8.0) + mask2
        m = jnp.max(s, axis=-1, keepdims=True)
        e = jnp.exp(s - m)
        a = e / jnp.sum(e, axis=-1, keepdims=True)
        av = jnp.dot(a, vs[kv], preferred_element_type=jnp.float32,
                     precision=HP)
        for g in range(G):
            outs.append(av[g * T:(g + 1) * T, :])

    cat = jnp.concatenate(outs, axis=-1)
    o = jnp.dot(cat, o_w_ref[...], preferred_element_type=jnp.float32,
                precision=HP)
    res2 = x + o
    res2_ref[...] = res2
    flat_ref[...] = _rms(res2, wpost_ref[...])


def _attention(hidden_states, cos, sin, mask, win, wpost, qln, kln, qkv_w, o_w):
    out_shape = [jax.ShapeDtypeStruct((T, H), jnp.float32),
                 jax.ShapeDtypeStruct((T, H), jnp.float32)]
    full = lambda shape: pl.BlockSpec(shape, lambda: tuple(0 for _ in shape))
    return pl.pallas_call(
        _attn_kernel,
        in_specs=[
            full((T, H)), full((T, HD)), full((T, HD)), full((T, T)),
            full((1, H)), full((1, H)), full((1, HD)), full((1, HD)),
            full((H, HD * (NH + 2 * NKV))), full((NH * HD, H)),
        ],
        out_specs=[full((T, H)), full((T, H))],
        out_shape=out_shape,
    )(hidden_states, cos, sin, mask, win, wpost, qln, kln, qkv_w, o_w)


def _routing_kernel(flat_ref, wg_ref, d_ref, c_ref):
    flat = flat_ref[...]
    logits = jnp.dot(flat, wg_ref[...], preferred_element_type=jnp.float32,
                     precision=jax.lax.Precision.HIGHEST)
    m = jnp.max(logits, axis=-1, keepdims=True)
    ex = jnp.exp(logits - m)
    gates = ex / jnp.sum(ex, axis=-1, keepdims=True)

    lane_e = jax.lax.broadcasted_iota(jnp.int32, (T, E), 1)
    work = gates
    idx_cols = []
    w_cols = []
    for _ in range(TOPK):
        mx = jnp.max(work, axis=-1, keepdims=True)
        eq = work == mx
        idx = jnp.min(jnp.where(eq, lane_e, E + 1), axis=-1, keepdims=True)
        idx_cols.append(idx)
        w_cols.append(mx)
        work = jnp.where(lane_e == idx, -1e30, work)

    wsum = w_cols[0]
    for j in range(1, TOPK):
        wsum = wsum + w_cols[j]
    denom = jnp.maximum(wsum, 1.1920929e-07)
    w_cols = [w / denom for w in w_cols]

    cnt = jnp.zeros((T, E), jnp.float32)
    ohs = []
    for j in range(TOPK):
        oh = (lane_e == idx_cols[j]).astype(jnp.float32)
        ohs.append(oh)
        cnt = cnt + oh
    r = jax.lax.broadcasted_iota(jnp.int32, (T, T), 0)
    c = jax.lax.broadcasted_iota(jnp.int32, (T, T), 1)
    lstrict = (r > c).astype(jnp.float32)
    base_excl = jnp.dot(lstrict, cnt, preferred_element_type=jnp.float32)

    slot_cols = []
    for j in range(TOPK):
        intra = jnp.zeros((T, 1), jnp.float32)
        for j2 in range(j):
            intra = intra + (idx_cols[j2] == idx_cols[j]).astype(jnp.float32)
        pos = jnp.sum(base_excl * ohs[j], axis=-1, keepdims=True) + intra
        slot_cols.append(jnp.where(
            pos < CAP, idx_cols[j].astype(jnp.float32) * CAP + pos, -1.0))

    slot = jnp.concatenate(slot_cols, axis=-1)
    wk = jnp.concatenate(w_cols, axis=-1)

    i8r = jax.lax.broadcasted_iota(jnp.int32, (TOPK, TOPK), 0)
    i8c = jax.lax.broadcasted_iota(jnp.int32, (TOPK, TOPK), 1)
    eye8 = (i8r == i8c).astype(jnp.float32)
    tr = lambda a: jax.lax.dot_general(eye8, a, (((1,), (1,)), ((), ())),
                                       preferred_element_type=jnp.float32,
                                       precision=jax.lax.Precision.HIGHEST)
    d_ref[...] = tr(slot)
    c_ref[...] = tr(wk)


def _routing(flat, wg):
    out_shape = [jax.ShapeDtypeStruct((TOPK, T), jnp.float32),
                 jax.ShapeDtypeStruct((TOPK, T), jnp.float32)]
    return pl.pallas_call(
        _routing_kernel,
        in_specs=[pl.BlockSpec((T, H), lambda: (0, 0)),
                  pl.BlockSpec((H, E), lambda: (0, 0))],
        out_specs=[pl.BlockSpec((TOPK, T), lambda: (0, 0)),
                   pl.BlockSpec((TOPK, T), lambda: (0, 0))],
        out_shape=out_shape,
    )(flat, wg)


NSF = 4
SFB = FH // NSF


def _shared_kernel(flat_ref, res2_ref, gu1_ref, gu2_ref, dw_ref, base_ref):
    fs = pl.program_id(0)
    flat = flat_ref[...]
    g1 = jnp.dot(flat, gu1_ref[...], preferred_element_type=jnp.float32)
    g2 = jnp.dot(flat, gu2_ref[...], preferred_element_type=jnp.float32)
    act = g1 * (g2 * jax.nn.sigmoid(g2))
    y = jnp.dot(act, dw_ref[...], preferred_element_type=jnp.float32)

    @pl.when(fs == 0)
    def _():
        base_ref[...] = res2_ref[...] + y

    @pl.when(fs != 0)
    def _():
        base_ref[...] = base_ref[...] + y


def _shared(flat, res2, shared_gu, shared_dw):
    return pl.pallas_call(
        _shared_kernel,
        grid=(NSF,),
        in_specs=[
            pl.BlockSpec((T, H), lambda fs: (0, 0)),
            pl.BlockSpec((T, H), lambda fs: (0, 0)),
            pl.BlockSpec((H, SFB), lambda fs: (0, fs)),
            pl.BlockSpec((H, SFB), lambda fs: (0, fs + NSF)),
            pl.BlockSpec((SFB, H), lambda fs: (fs, 0)),
        ],
        out_specs=pl.BlockSpec((T, H), lambda fs: (0, 0)),
        out_shape=jax.ShapeDtypeStruct((T, H), jnp.float32),
    )(flat, res2, shared_gu, shared_gu, shared_dw)


NF = 2
FB = FH // NF


def _moe_kernel(slot_ref, wk_ref, flat_ref, base_ref, gu1_ref, gu2_ref, dw_ref,
                out_ref, buf_ref, c_ref):
    e = pl.program_id(0)
    f = pl.program_id(1)

    @pl.when(f == 0)
    def _():
        st = slot_ref[...].astype(jnp.int32) - e * CAP
        wk = wk_ref[...]
        ci = jax.lax.broadcasted_iota(jnp.int32, (CAP, T), 0)
        dm = jnp.zeros((CAP, T), jnp.float32)
        cm = jnp.zeros((CAP, T), jnp.float32)
        for j in range(TOPK):
            eqf = (ci == st[j:j + 1, :]).astype(jnp.float32)
            dm = dm + eqf
            cm = cm + eqf * wk[j:j + 1, :]
        c_ref[...] = cm
        buf_ref[...] = jnp.dot(dm, flat_ref[...],
                               preferred_element_type=jnp.float32)

    x = buf_ref[...]
    g1 = jnp.dot(x, gu1_ref[0], preferred_element_type=jnp.float32)
    g2 = jnp.dot(x, gu2_ref[0], preferred_element_type=jnp.float32)
    act = g1 * (g2 * jax.nn.sigmoid(g2))
    y = jnp.dot(act, dw_ref[0], preferred_element_type=jnp.float32)
    contrib = jax.lax.dot_general(c_ref[...], y, (((0,), (0,)), ((), ())),
                                  preferred_element_type=jnp.float32)

    @pl.when((e == 0) & (f == 0))
    def _():
        out_ref[...] = base_ref[...] + contrib

    @pl.when((e != 0) | (f != 0))
    def _():
        out_ref[...] = out_ref[...] + contrib


def _moe(slot_t, wk_t, flat, base, expert_gu, expert_dw):
    return pl.pallas_call(
        _moe_kernel,
        grid=(E, NF),
        in_specs=[
            pl.BlockSpec((TOPK, T), lambda e, f: (0, 0)),
            pl.BlockSpec((TOPK, T), lambda e, f: (0, 0)),
            pl.BlockSpec((T, H), lambda e, f: (0, 0)),
            pl.BlockSpec((T, H), lambda e, f: (0, 0)),
            pl.BlockSpec((1, H, FB), lambda e, f: (e, 0, f)),
            pl.BlockSpec((1, H, FB), lambda e, f: (e, 0, f + NF)),
            pl.BlockSpec((1, FB, H), lambda e, f: (e, f, 0)),
        ],
        out_specs=pl.BlockSpec((T, H), lambda e, f: (0, 0)),
        out_shape=jax.ShapeDtypeStruct((T, H), jnp.float32),
        scratch_shapes=[pltpu.VMEM((CAP, H), jnp.float32),
                        pltpu.VMEM((CAP, T), jnp.float32)],
    )(slot_t, wk_t, flat, base, expert_gu, expert_gu, expert_dw)


@jax.jit
def kernel(hidden_states, cos, sin, attention_mask, input_ln_w, post_ln_w,
           q_ln_w, k_ln_w, qkv_w, o_w, wg, shared_gu, shared_dw,
           expert_gu, expert_dw):
    cos_t = jnp.tile(cos[0], (B, 1))
    sin_t = jnp.tile(sin[0], (B, 1))
    ri = jnp.arange(T)
    same_batch = (ri[:, None] // S) == (ri[None, :] // S)
    mask_full = jnp.where(same_batch, jnp.tile(attention_mask[0, 0], (B, B)),
                          -1e9).astype(jnp.float32)
    res2, flat = _attention(
        hidden_states.reshape(T, H), cos_t, sin_t, mask_full,
        input_ln_w.reshape(1, H), post_ln_w.reshape(1, H),
        q_ln_w.reshape(1, HD), k_ln_w.reshape(1, HD), qkv_w, o_w)
    d, c = _routing(flat, wg)
    base = _shared(flat, res2, shared_gu, shared_dw)
    out = _moe(d, c, flat, base, expert_gu, expert_dw)
    return out.reshape(B, S, H)


# speedup vs baseline: 1.4061x; 1.0064x over previous
import functools

import jax
import jax.numpy as jnp
from jax.experimental import pallas as pl
from jax.experimental.pallas import tpu as pltpu

B, S, H = 8, 32, 1024
NH, NKV, HD = 16, 8, 64
G = NH // NKV
E, TOPK, CAP = 64, 8, 64
FF = 6144
FH = FF // 2
T = B * S
NSLOT = E * CAP


def _rms(x, w, eps=1e-6):
    v = jnp.mean(jnp.square(x), axis=-1, keepdims=True)
    return w * (x * jax.lax.rsqrt(v + eps))


def _rot_half(x):
    h = x.shape[-1] // 2
    return jnp.concatenate([-x[:, h:], x[:, :h]], axis=-1)


HI = jax.lax.Precision.HIGHEST
HP = jax.lax.Precision.DEFAULT


def _attn_kernel(hs_ref, cos_ref, sin_ref, win_ref, wpost_ref,
                 qln_ref, kln_ref, qkv_w_ref, o_w_ref, res2_ref, flat_ref):
    x = hs_ref[...]
    xn = _rms(x, win_ref[...])
    qkv = jnp.dot(xn, qkv_w_ref[...], preferred_element_type=jnp.float32,
                  precision=HP)
    cos = cos_ref[...]
    sin = sin_ref[...]
    qln = qln_ref[...]
    kln = kln_ref[...]

    ri = jax.lax.broadcasted_iota(jnp.int32, (G * T, T), 0)
    ci = jax.lax.broadcasted_iota(jnp.int32, (G * T, T), 1)
    same = ((ri // S) % B) == (ci // S)
    causal = (ri % S) >= (ci % S)
    mask2 = jnp.where(same & causal, 0.0, -1e9)

    ks = []
    vs = []
    for kv in range(NKV):
        base = kv * (G + 2) * HD
        k = qkv[:, base + G * HD: base + (G + 1) * HD]
        v = qkv[:, base + (G + 1) * HD: base + (G + 2) * HD]
        k = k * cos + _rot_half(k) * sin
        k = _rms(k, kln)
        ks.append(k)
        vs.append(v)

    outs = []
    for kv in range(NKV):
        base = kv * (G + 2) * HD
        qs = []
        for g in range(G):
            q = qkv[:, base + g * HD: base + (g + 1) * HD]
            q = q * cos + _rot_half(q) * sin
            qs.append(_rms(q, qln))
        qpair = jnp.concatenate(qs, axis=0)
        s = jax.lax.dot_general(qpair, ks[kv], (((1,), (1,)), ((), ())),
                                preferred_element_type=jnp.float32,
                                precision=HP)
        s = s * (1.0 / 8.0) + mask2
        m = jnp.max(s, axis=-1, keepdims=True)
        e = jnp.exp(s - m)
        a = e / jnp.sum(e, axis=-1, keepdims=True)
        av = jnp.dot(a, vs[kv], preferred_element_type=jnp.float32,
                     precision=HP)
        for g in range(G):
            outs.append(av[g * T:(g + 1) * T, :])

    cat = jnp.concatenate(outs, axis=-1)
    o = jnp.dot(cat, o_w_ref[...], preferred_element_type=jnp.float32,
                precision=HP)
    res2 = x + o
    res2_ref[...] = res2
    flat_ref[...] = _rms(res2, wpost_ref[...])


def _attention(hidden_states, cos, sin, win, wpost, qln, kln, qkv_w, o_w):
    out_shape = [jax.ShapeDtypeStruct((T, H), jnp.float32),
                 jax.ShapeDtypeStruct((T, H), jnp.float32)]
    full = lambda shape: pl.BlockSpec(shape, lambda: tuple(0 for _ in shape))
    return pl.pallas_call(
        _attn_kernel,
        in_specs=[
            full((T, H)), full((T, HD)), full((T, HD)),
            full((1, H)), full((1, H)), full((1, HD)), full((1, HD)),
            full((H, HD * (NH + 2 * NKV))), full((NH * HD, H)),
        ],
        out_specs=[full((T, H)), full((T, H))],
        out_shape=out_shape,
    )(hidden_states, cos, sin, win, wpost, qln, kln, qkv_w, o_w)


def _routing_kernel(flat_ref, wg_ref, d_ref, c_ref):
    flat = flat_ref[...]
    logits = jnp.dot(flat, wg_ref[...], preferred_element_type=jnp.float32,
                     precision=jax.lax.Precision.HIGHEST)
    m = jnp.max(logits, axis=-1, keepdims=True)
    ex = jnp.exp(logits - m)
    gates = ex / jnp.sum(ex, axis=-1, keepdims=True)

    lane_e = jax.lax.broadcasted_iota(jnp.int32, (T, E), 1)
    work = gates
    idx_cols = []
    w_cols = []
    for _ in range(TOPK):
        mx = jnp.max(work, axis=-1, keepdims=True)
        eq = work == mx
        idx = jnp.min(jnp.where(eq, lane_e, E + 1), axis=-1, keepdims=True)
        idx_cols.append(idx)
        w_cols.append(mx)
        work = jnp.where(lane_e == idx, -1e30, work)

    wsum = w_cols[0]
    for j in range(1, TOPK):
        wsum = wsum + w_cols[j]
    denom = jnp.maximum(wsum, 1.1920929e-07)
    w_cols = [w / denom for w in w_cols]

    cnt = jnp.zeros((T, E), jnp.float32)
    ohs = []
    for j in range(TOPK):
        oh = (lane_e == idx_cols[j]).astype(jnp.float32)
        ohs.append(oh)
        cnt = cnt + oh
    r = jax.lax.broadcasted_iota(jnp.int32, (T, T), 0)
    c = jax.lax.broadcasted_iota(jnp.int32, (T, T), 1)
    lstrict = (r > c).astype(jnp.float32)
    base_excl = jnp.dot(lstrict, cnt, preferred_element_type=jnp.float32)

    slot_cols = []
    for j in range(TOPK):
        intra = jnp.zeros((T, 1), jnp.float32)
        for j2 in range(j):
            intra = intra + (idx_cols[j2] == idx_cols[j]).astype(jnp.float32)
        pos = jnp.sum(base_excl * ohs[j], axis=-1, keepdims=True) + intra
        slot_cols.append(jnp.where(
            pos < CAP, idx_cols[j].astype(jnp.float32) * CAP + pos, -1.0))

    slot = jnp.concatenate(slot_cols, axis=-1)
    wk = jnp.concatenate(w_cols, axis=-1)

    i8r = jax.lax.broadcasted_iota(jnp.int32, (TOPK, TOPK), 0)
    i8c = jax.lax.broadcasted_iota(jnp.int32, (TOPK, TOPK), 1)
    eye8 = (i8r == i8c).astype(jnp.float32)
    tr = lambda a: jax.lax.dot_general(eye8, a, (((1,), (1,)), ((), ())),
                                       preferred_element_type=jnp.float32,
                                       precision=jax.lax.Precision.HIGHEST)
    d_ref[...] = tr(slot)
    c_ref[...] = tr(wk)


def _routing(flat, wg):
    out_shape = [jax.ShapeDtypeStruct((TOPK, T), jnp.float32),
                 jax.ShapeDtypeStruct((TOPK, T), jnp.float32)]
    return pl.pallas_call(
        _routing_kernel,
        in_specs=[pl.BlockSpec((T, H), lambda: (0, 0)),
                  pl.BlockSpec((H, E), lambda: (0, 0))],
        out_specs=[pl.BlockSpec((TOPK, T), lambda: (0, 0)),
                   pl.BlockSpec((TOPK, T), lambda: (0, 0))],
        out_shape=out_shape,
    )(flat, wg)


NSF = 4
SFB = FH // NSF


def _shared_kernel(flat_ref, res2_ref, gu1_ref, gu2_ref, dw_ref, base_ref):
    fs = pl.program_id(0)
    flat = flat_ref[...]
    g1 = jnp.dot(flat, gu1_ref[...], preferred_element_type=jnp.float32)
    g2 = jnp.dot(flat, gu2_ref[...], preferred_element_type=jnp.float32)
    act = g1 * (g2 * jax.nn.sigmoid(g2))
    y = jnp.dot(act, dw_ref[...], preferred_element_type=jnp.float32)

    @pl.when(fs == 0)
    def _():
        base_ref[...] = res2_ref[...] + y

    @pl.when(fs != 0)
    def _():
        base_ref[...] = base_ref[...] + y


def _shared(flat, res2, shared_gu, shared_dw):
    return pl.pallas_call(
        _shared_kernel,
        grid=(NSF,),
        in_specs=[
            pl.BlockSpec((T, H), lambda fs: (0, 0)),
            pl.BlockSpec((T, H), lambda fs: (0, 0)),
            pl.BlockSpec((H, SFB), lambda fs: (0, fs)),
            pl.BlockSpec((H, SFB), lambda fs: (0, fs + NSF)),
            pl.BlockSpec((SFB, H), lambda fs: (fs, 0)),
        ],
        out_specs=pl.BlockSpec((T, H), lambda fs: (0, 0)),
        out_shape=jax.ShapeDtypeStruct((T, H), jnp.float32),
    )(flat, res2, shared_gu, shared_gu, shared_dw)


NF = 2
FB = FH // NF


def _moe_kernel(slot_ref, wk_ref, flat_ref, base_ref, gu1_ref, gu2_ref, dw_ref,
                out_ref, buf_ref, c_ref):
    e = pl.program_id(0)
    f = pl.program_id(1)

    @pl.when(f == 0)
    def _():
        st = slot_ref[...].astype(jnp.int32) - e * CAP
        wk = wk_ref[...]
        ci = jax.lax.broadcasted_iota(jnp.int32, (CAP, T), 0)
        dm = jnp.zeros((CAP, T), jnp.float32)
        cm = jnp.zeros((CAP, T), jnp.float32)
        for j in range(TOPK):
            eqf = (ci == st[j:j + 1, :]).astype(jnp.float32)
            dm = dm + eqf
            cm = cm + eqf * wk[j:j + 1, :]
        c_ref[...] = cm
        buf_ref[...] = jnp.dot(dm, flat_ref[...],
                               preferred_element_type=jnp.float32)

    x = buf_ref[...]
    g1 = jnp.dot(x, gu1_ref[0], preferred_element_type=jnp.float32)
    g2 = jnp.dot(x, gu2_ref[0], preferred_element_type=jnp.float32)
    act = g1 * (g2 * jax.nn.sigmoid(g2))
    y = jnp.dot(act, dw_ref[0], preferred_element_type=jnp.float32)
    contrib = jax.lax.dot_general(c_ref[...], y, (((0,), (0,)), ((), ())),
                                  preferred_element_type=jnp.float32)

    @pl.when((e == 0) & (f == 0))
    def _():
        out_ref[...] = base_ref[...] + contrib

    @pl.when((e != 0) | (f != 0))
    def _():
        out_ref[...] = out_ref[...] + contrib


def _moe(slot_t, wk_t, flat, base, expert_gu, expert_dw):
    return pl.pallas_call(
        _moe_kernel,
        grid=(E, NF),
        in_specs=[
            pl.BlockSpec((TOPK, T), lambda e, f: (0, 0)),
            pl.BlockSpec((TOPK, T), lambda e, f: (0, 0)),
            pl.BlockSpec((T, H), lambda e, f: (0, 0)),
            pl.BlockSpec((T, H), lambda e, f: (0, 0)),
            pl.BlockSpec((1, H, FB), lambda e, f: (e, 0, f)),
            pl.BlockSpec((1, H, FB), lambda e, f: (e, 0, f + NF)),
            pl.BlockSpec((1, FB, H), lambda e, f: (e, f, 0)),
        ],
        out_specs=pl.BlockSpec((T, H), lambda e, f: (0, 0)),
        out_shape=jax.ShapeDtypeStruct((T, H), jnp.float32),
        scratch_shapes=[pltpu.VMEM((CAP, H), jnp.float32),
                        pltpu.VMEM((CAP, T), jnp.float32)],
    )(slot_t, wk_t, flat, base, expert_gu, expert_gu, expert_dw)


@jax.jit
def kernel(hidden_states, cos, sin, attention_mask, input_ln_w, post_ln_w,
           q_ln_w, k_ln_w, qkv_w, o_w, wg, shared_gu, shared_dw,
           expert_gu, expert_dw):
    cos_t = jnp.tile(cos[0], (B, 1))
    sin_t = jnp.tile(sin[0], (B, 1))
    res2, flat = _attention(
        hidden_states.reshape(T, H), cos_t, sin_t,
        input_ln_w.reshape(1, H), post_ln_w.reshape(1, H),
        q_ln_w.reshape(1, HD), k_ln_w.reshape(1, HD), qkv_w, o_w)
    d, c = _routing(flat, wg)
    base = _shared(flat, res2, shared_gu, shared_dw)
    out = _moe(d, c, flat, base, expert_gu, expert_dw)
    return out.reshape(B, S, H)
